# baseline (speedup 1.0000x reference)
import jax
import jax.numpy as jnp
from jax.experimental import pallas as pl
from jax.experimental.pallas import tpu as pltpu

B, DIM, N = 8, 256, 1024
NQ, K = 4, 1024
TILE_N = 512
NT = N // TILE_N
SPLIT = 2
BH = B // SPLIT


def _rvq_body(x_ref, cb_ref, cbh_ref, cbm_ref, cbl_ref, qout_ref, louts_ref,
              idx_ref, loss_ref):
    res = x_ref[0]
    qout = jnp.zeros((DIM, TILE_N), jnp.float32)
    row_iota = jax.lax.broadcasted_iota(
        jnp.int32, (K, TILE_N), 0).astype(jnp.float32)
    for q in range(NQ):
        cb = cb_ref[q]
        cnorm = jnp.sum(cb * cb, axis=1, keepdims=True)
        rnorm = jnp.sum(res * res, axis=0, keepdims=True)
        scores = (rnorm - 2.0 * jnp.dot(cb, res)) + cnorm
        mins = jnp.min(scores, axis=0, keepdims=True)
        cand = jnp.where(scores == mins, row_iota, float(K))
        idxf = jnp.min(cand, axis=0)
        idx = idxf.astype(jnp.int32)
        onehot = (row_iota == idxf[None, :]).astype(jnp.bfloat16)
        dims = (((1,), (0,)), ((), ()))
        quant = (
            jax.lax.dot_general(cbh_ref[q], onehot, dims,
                                preferred_element_type=jnp.float32)
            + jax.lax.dot_general(cbm_ref[q], onehot, dims,
                                  preferred_element_type=jnp.float32)
            + jax.lax.dot_general(cbl_ref[q], onehot, dims,
                                  preferred_element_type=jnp.float32)
        )
        res = res - quant
        qout = qout + quant
        louts_ref[q, 0] = qout
        idx_ref[0, q, :] = idx
        loss_ref[0, 0, q : q + 1, :] = jnp.full((1, 128), jnp.sum(res * res))
    qout_ref[0] = qout


def _trunc_bf16(v):
    u = jax.lax.bitcast_convert_type(v, jnp.uint32)
    return jax.lax.bitcast_convert_type(u & jnp.uint32(0xFFFF0000), jnp.float32)


def _rvq_half(xh, codebooks, cb_hi, cb_mid, cb_lo):
    return pl.pallas_call(
        _rvq_body,
        grid=(BH, NT),
        in_specs=[
            pl.BlockSpec((1, DIM, TILE_N), lambda b, t: (b, 0, t)),
            pl.BlockSpec((NQ, K, DIM), lambda b, t: (0, 0, 0)),
            pl.BlockSpec((NQ, DIM, K), lambda b, t: (0, 0, 0)),
            pl.BlockSpec((NQ, DIM, K), lambda b, t: (0, 0, 0)),
            pl.BlockSpec((NQ, DIM, K), lambda b, t: (0, 0, 0)),
        ],
        out_specs=[
            pl.BlockSpec((1, DIM, TILE_N), lambda b, t: (b, 0, t)),
            pl.BlockSpec((NQ, 1, DIM, TILE_N), lambda b, t: (0, b, 0, t)),
            pl.BlockSpec((1, NQ, TILE_N), lambda b, t: (b, 0, t)),
            pl.BlockSpec((1, 1, NQ, 128), lambda b, t: (b, t, 0, 0)),
        ],
        out_shape=[
            jax.ShapeDtypeStruct((BH, DIM, N), jnp.float32),
            jax.ShapeDtypeStruct((NQ, BH, DIM, N), jnp.float32),
            jax.ShapeDtypeStruct((BH, NQ, N), jnp.int32),
            jax.ShapeDtypeStruct((BH, NT, NQ, 128), jnp.float32),
        ],
        compiler_params=pltpu.CompilerParams(
            dimension_semantics=("parallel", "parallel"),
        ),
    )(xh, codebooks, cb_hi, cb_mid, cb_lo)


@jax.jit
def kernel(x, codebooks):
    hi_f = _trunc_bf16(codebooks)
    r1 = codebooks - hi_f
    mid_f = _trunc_bf16(r1)
    lo_f = r1 - mid_f
    cb_hi = jnp.transpose(hi_f.astype(jnp.bfloat16), (0, 2, 1))
    cb_mid = jnp.transpose(mid_f.astype(jnp.bfloat16), (0, 2, 1))
    cb_lo = jnp.transpose(lo_f.astype(jnp.bfloat16), (0, 2, 1))
    parts = [
        _rvq_half(x[i * BH : (i + 1) * BH], codebooks, cb_hi, cb_mid, cb_lo)
        for i in range(SPLIT)
    ]
    qout = jnp.concatenate([p[0] for p in parts], axis=0)
    all_indices = jnp.concatenate(
        [jnp.transpose(p[2], (0, 2, 1)) for p in parts], axis=0)
    loss_sum = sum(jnp.sum(p[3][:, :, :, 0], axis=(0, 1)) for p in parts)
    all_losses = loss_sum / (B * N * DIM)
    layer_out = jnp.concatenate(
        [jnp.transpose(p[1], (1, 2, 3, 0)) for p in parts], axis=0)
    return qout, all_indices, all_losses, layer_out


# speedup vs baseline: 1.3594x; 1.3594x over previous
import jax
import jax.numpy as jnp
from jax.experimental import pallas as pl
from jax.experimental.pallas import tpu as pltpu

B, DIM, N = 8, 256, 1024
NQ, K = 4, 1024
TILE_N = 1024
NT = N // TILE_N


def _rvq_body(x_ref, cb_ref, cbh_ref, cbm_ref, cbl_ref, qout_ref, louts_ref,
              idx_ref, loss_ref):
    res = x_ref[0]
    qout = jnp.zeros((DIM, TILE_N), jnp.float32)
    row_iota = jax.lax.broadcasted_iota(
        jnp.int32, (K, TILE_N), 0).astype(jnp.float32)
    rnorm = jnp.sum(res * res, axis=0, keepdims=True)
    for q in range(NQ):
        cb = cb_ref[q]
        cnorm = jnp.sum(cb * cb, axis=1, keepdims=True)
        scores = (rnorm - 2.0 * jnp.dot(cb, res)) + cnorm
        mins = jnp.min(scores, axis=0, keepdims=True)
        cand = jnp.where(scores == mins, row_iota, float(K))
        idxf = jnp.min(cand, axis=0)
        idx = idxf.astype(jnp.int32)
        onehot = (row_iota == idxf[None, :]).astype(jnp.bfloat16)
        dims = (((1,), (0,)), ((), ()))
        quant = (
            jax.lax.dot_general(cbh_ref[q], onehot, dims,
                                preferred_element_type=jnp.float32)
            + jax.lax.dot_general(cbm_ref[q], onehot, dims,
                                  preferred_element_type=jnp.float32)
            + jax.lax.dot_general(cbl_ref[q], onehot, dims,
                                  preferred_element_type=jnp.float32)
        )
        res = res - quant
        qout = qout + quant
        rnorm = jnp.sum(res * res, axis=0, keepdims=True)
        louts_ref[q, 0] = qout
        idx_ref[0, q, :] = idx
        loss_ref[0, 0, q : q + 1, :] = jnp.full((1, 128), jnp.sum(rnorm))
    qout_ref[0] = qout


def _trunc_bf16(v):
    u = jax.lax.bitcast_convert_type(v, jnp.uint32)
    return jax.lax.bitcast_convert_type(u & jnp.uint32(0xFFFF0000), jnp.float32)


@jax.jit
def kernel(x, codebooks):
    hi_f = _trunc_bf16(codebooks)
    r1 = codebooks - hi_f
    mid_f = _trunc_bf16(r1)
    lo_f = r1 - mid_f
    cb_hi = jnp.transpose(hi_f.astype(jnp.bfloat16), (0, 2, 1))
    cb_mid = jnp.transpose(mid_f.astype(jnp.bfloat16), (0, 2, 1))
    cb_lo = jnp.transpose(lo_f.astype(jnp.bfloat16), (0, 2, 1))
    qout, louts, idx_out, loss_parts = pl.pallas_call(
        _rvq_body,
        grid=(B, NT),
        in_specs=[
            pl.BlockSpec((1, DIM, TILE_N), lambda b, t: (b, 0, t)),
            pl.BlockSpec((NQ, K, DIM), lambda b, t: (0, 0, 0)),
            pl.BlockSpec((NQ, DIM, K), lambda b, t: (0, 0, 0)),
            pl.BlockSpec((NQ, DIM, K), lambda b, t: (0, 0, 0)),
            pl.BlockSpec((NQ, DIM, K), lambda b, t: (0, 0, 0)),
        ],
        out_specs=[
            pl.BlockSpec((1, DIM, TILE_N), lambda b, t: (b, 0, t)),
            pl.BlockSpec((NQ, 1, DIM, TILE_N), lambda b, t: (0, b, 0, t)),
            pl.BlockSpec((1, NQ, TILE_N), lambda b, t: (b, 0, t)),
            pl.BlockSpec((1, 1, NQ, 128), lambda b, t: (b, t, 0, 0)),
        ],
        out_shape=[
            jax.ShapeDtypeStruct((B, DIM, N), jnp.float32),
            jax.ShapeDtypeStruct((NQ, B, DIM, N), jnp.float32),
            jax.ShapeDtypeStruct((B, NQ, N), jnp.int32),
            jax.ShapeDtypeStruct((B, NT, NQ, 128), jnp.float32),
        ],
        compiler_params=pltpu.CompilerParams(
            dimension_semantics=("parallel", "parallel"),
        ),
    )(x, codebooks, cb_hi, cb_mid, cb_lo)
    all_indices = jnp.transpose(idx_out, (0, 2, 1))
    all_losses = jnp.sum(loss_parts[:, :, :, 0], axis=(0, 1)) / (B * N * DIM)
    layer_out = jnp.transpose(louts, (1, 2, 3, 0))
    return qout, all_indices, all_losses, layer_out


# speedup vs baseline: 1.4434x; 1.0618x over previous
import jax
import jax.numpy as jnp
from jax.experimental import pallas as pl
from jax.experimental.pallas import tpu as pltpu

B, DIM, N = 8, 256, 1024
NQ, K = 4, 1024
TILE_N = 1024
NT = N // TILE_N


def _rvq_body(x_ref, cb_ref, cbh_ref, cbm_ref, cbl_ref, qout_ref, louts_ref,
              idx_ref, loss_ref):
    res = x_ref[0]
    qout = jnp.zeros((DIM, TILE_N), jnp.float32)
    row_iota = jax.lax.broadcasted_iota(jnp.int32, (K, TILE_N), 0)
    rnorm = jnp.sum(res * res, axis=0, keepdims=True)
    for q in range(NQ):
        cb = cb_ref[q]
        cnorm = jnp.sum(cb * cb, axis=1, keepdims=True)
        scores = (rnorm - 2.0 * jnp.dot(cb, res)) + cnorm
        idx = jnp.argmin(scores, axis=0)
        onehot = (row_iota == idx[None, :]).astype(jnp.bfloat16)
        dims = (((1,), (0,)), ((), ()))
        quant = (
            jax.lax.dot_general(cbh_ref[q], onehot, dims,
                                preferred_element_type=jnp.float32)
            + jax.lax.dot_general(cbm_ref[q], onehot, dims,
                                  preferred_element_type=jnp.float32)
            + jax.lax.dot_general(cbl_ref[q], onehot, dims,
                                  preferred_element_type=jnp.float32)
        )
        res = res - quant
        qout = qout + quant
        rnorm = jnp.sum(res * res, axis=0, keepdims=True)
        louts_ref[q, 0] = qout
        idx_ref[0, q, :] = idx
        loss_ref[0, 0, q : q + 1, :] = jnp.full((1, 128), jnp.sum(rnorm))
    qout_ref[0] = qout


def _trunc_bf16(v):
    u = jax.lax.bitcast_convert_type(v, jnp.uint32)
    return jax.lax.bitcast_convert_type(u & jnp.uint32(0xFFFF0000), jnp.float32)


@jax.jit
def kernel(x, codebooks):
    cbt = jnp.transpose(codebooks, (0, 2, 1))
    hi_f = _trunc_bf16(cbt)
    r1 = cbt - hi_f
    mid_f = _trunc_bf16(r1)
    lo_f = r1 - mid_f
    cb_hi = hi_f.astype(jnp.bfloat16)
    cb_mid = mid_f.astype(jnp.bfloat16)
    cb_lo = lo_f.astype(jnp.bfloat16)
    qout, louts, idx_out, loss_parts = pl.pallas_call(
        _rvq_body,
        grid=(B, NT),
        in_specs=[
            pl.BlockSpec((1, DIM, TILE_N), lambda b, t: (b, 0, t)),
            pl.BlockSpec((NQ, K, DIM), lambda b, t: (0, 0, 0)),
            pl.BlockSpec((NQ, DIM, K), lambda b, t: (0, 0, 0)),
            pl.BlockSpec((NQ, DIM, K), lambda b, t: (0, 0, 0)),
            pl.BlockSpec((NQ, DIM, K), lambda b, t: (0, 0, 0)),
        ],
        out_specs=[
            pl.BlockSpec((1, DIM, TILE_N), lambda b, t: (b, 0, t)),
            pl.BlockSpec((NQ, 1, DIM, TILE_N), lambda b, t: (0, b, 0, t)),
            pl.BlockSpec((1, NQ, TILE_N), lambda b, t: (b, 0, t)),
            pl.BlockSpec((1, 1, NQ, 128), lambda b, t: (b, t, 0, 0)),
        ],
        out_shape=[
            jax.ShapeDtypeStruct((B, DIM, N), jnp.float32),
            jax.ShapeDtypeStruct((NQ, B, DIM, N), jnp.float32),
            jax.ShapeDtypeStruct((B, NQ, N), jnp.int32),
            jax.ShapeDtypeStruct((B, NT, NQ, 128), jnp.float32),
        ],
        compiler_params=pltpu.CompilerParams(
            dimension_semantics=("parallel", "parallel"),
        ),
    )(x, codebooks, cb_hi, cb_mid, cb_lo)
    all_indices = jnp.transpose(idx_out, (0, 2, 1))
    all_losses = jnp.sum(loss_parts[:, :, :, 0], axis=(0, 1)) / (B * N * DIM)
    layer_out = jnp.transpose(louts, (1, 2, 3, 0))
    return qout, all_indices, all_losses, layer_out
